# baseline (speedup 1.0000x reference)
import jax
import jax.numpy as jnp
from jax import lax
from jax.experimental import pallas as pl
from jax.experimental.pallas import tpu as pltpu
from jax.experimental.pallas import tpu_sc as plsc

VOCAB = 100000
EMB = 128
SEQ = 200
BATCH = 1024

NC = 2
NS = 16
NW = NC * NS

ROWS = BATCH * SEQ
ROWS_PER_W = ROWS // NW
CHUNK = 100
CHUNKS_PER_W = ROWS_PER_W // CHUNK
NBUF = 4
GROUPS = CHUNKS_PER_W // NBUF


def _body(table_hbm, tokens_hbm, pos_hbm, out_hbm, idx_v, rows4, *sems):
    psem = sems[0:NBUF]
    gsem = sems[NBUF:2 * NBUF]
    osem = sems[2 * NBUF:3 * NBUF]
    wid = lax.axis_index("s") * NC + lax.axis_index("c")
    out_base = wid * ROWS_PER_W

    pltpu.sync_copy(tokens_hbm.at[pl.ds(wid * CHUNKS_PER_W, CHUNKS_PER_W)], idx_v)

    def prefill(c, u):
        half = lax.rem(c, 2)
        pltpu.async_copy(pos_hbm.at[pl.ds(half * CHUNK, CHUNK)], rows4.at[u], psem[u])

    prefill(0, 0)

    def group_step(g, carry):
        for u in range(NBUF):
            c = g * NBUF + u
            u1 = (u + 1) % NBUF
            pltpu.make_async_copy(
                pos_hbm.at[pl.ds(0, CHUNK)], rows4.at[u], psem[u]).wait()
            gd = pltpu.async_copy(
                table_hbm.at[idx_v.at[c]], rows4.at[u], gsem[u], add=True)
            @pl.when(c >= NBUF - 1)
            def _():
                pltpu.make_async_copy(
                    rows4.at[u1], out_hbm.at[pl.ds(0, CHUNK)], osem[u1]).wait()

            @pl.when(c + 1 < CHUNKS_PER_W)
            def _():
                prefill(c + 1, u1)

            gd.wait()
            pltpu.async_copy(
                rows4.at[u], out_hbm.at[pl.ds(out_base + c * CHUNK, CHUNK)], osem[u])
        return carry

    lax.fori_loop(0, GROUPS, group_step, 0)

    for u in range(1, NBUF):
        pltpu.make_async_copy(
            rows4.at[u], out_hbm.at[pl.ds(0, CHUNK)], osem[u]).wait()


@jax.jit
def _emb(tokens2d, table, pos):
    mesh = plsc.VectorSubcoreMesh(core_axis_name="c", subcore_axis_name="s")
    k = pl.kernel(
        _body,
        out_type=jax.ShapeDtypeStruct((ROWS, EMB), jnp.float32),
        mesh=mesh,
        scratch_types=[
            pltpu.VMEM((CHUNKS_PER_W, CHUNK), jnp.int32),
            pltpu.VMEM((NBUF, CHUNK, EMB), jnp.float32),
        ] + [pltpu.SemaphoreType.DMA] * (3 * NBUF),
        compiler_params=pltpu.CompilerParams(use_tc_tiling_on_sc=False),
    )
    return k(table, tokens2d, pos)


def kernel(tokens, token_embedding, positional_embedding):
    tokens2d = tokens.astype(jnp.int32).reshape(ROWS // CHUNK, CHUNK)
    out = _emb(tokens2d, token_embedding, positional_embedding)
    return out.reshape(BATCH, SEQ, EMB)


# speedup vs baseline: 2.2736x; 2.2736x over previous
import jax
import jax.numpy as jnp
from jax import lax
from jax.experimental import pallas as pl
from jax.experimental.pallas import tpu as pltpu
from jax.experimental.pallas import tpu_sc as plsc

VOCAB = 100000
EMB = 128
SEQ = 200
BATCH = 1024

NC = 2
NS = 16
NW = NC * NS

ROWS = BATCH * SEQ
ROWS_PER_W = ROWS // NW
CHUNK = 100
CHUNKS_PER_W = ROWS_PER_W // CHUNK
NBUF = 4
GROUPS = CHUNKS_PER_W // NBUF
LANES = 16
VECS_PER_ROW = EMB // LANES


def _body(table_hbm, tokens_hbm, pos_hbm, out_hbm, idx_v, pos_v, rows4, *sems):
    gsem = sems[0:NBUF]
    osem = sems[NBUF:2 * NBUF]
    wid = lax.axis_index("s") * NC + lax.axis_index("c")
    out_base = wid * ROWS_PER_W

    pltpu.sync_copy(tokens_hbm.at[pl.ds(wid * CHUNKS_PER_W, CHUNKS_PER_W)], idx_v)
    pltpu.sync_copy(pos_hbm, pos_v)

    def gather(c, u):
        pltpu.async_copy(table_hbm.at[idx_v.at[c]], rows4.at[u], gsem[u])

    gather(0, 0)

    def group_step(g, carry):
        for u in range(NBUF):
            c = g * NBUF + u
            u1 = (u + 1) % NBUF
            rows_u = rows4.at[u]
            pltpu.make_async_copy(
                table_hbm.at[idx_v.at[c]], rows_u, gsem[u]).wait()
            @pl.when(c >= NBUF - 1)
            def _():
                pltpu.make_async_copy(
                    rows4.at[u1], out_hbm.at[pl.ds(0, CHUNK)], osem[u1]).wait()

            @pl.when(c + 1 < CHUNKS_PER_W)
            def _():
                gather(c + 1, u1)

            pr0 = lax.rem(c, 2) * CHUNK

            def add_row(r, carry2):
                pr = pr0 + r
                for d in range(VECS_PER_ROW):
                    sl = pl.ds(d * LANES, LANES)
                    plsc.addupdate(rows_u.at[r, sl], pos_v[pr, sl])
                return carry2

            lax.fori_loop(0, CHUNK, add_row, 0)
            pltpu.async_copy(
                rows_u, out_hbm.at[pl.ds(out_base + c * CHUNK, CHUNK)], osem[u])
        return carry

    lax.fori_loop(0, GROUPS, group_step, 0)

    for u in range(1, NBUF):
        pltpu.make_async_copy(
            rows4.at[u], out_hbm.at[pl.ds(0, CHUNK)], osem[u]).wait()


@jax.jit
def _emb(tokens2d, table, pos):
    mesh = plsc.VectorSubcoreMesh(core_axis_name="c", subcore_axis_name="s")
    k = pl.kernel(
        _body,
        out_type=jax.ShapeDtypeStruct((ROWS, EMB), jnp.float32),
        mesh=mesh,
        scratch_types=[
            pltpu.VMEM((CHUNKS_PER_W, CHUNK), jnp.int32),
            pltpu.VMEM((SEQ, EMB), jnp.float32),
            pltpu.VMEM((NBUF, CHUNK, EMB), jnp.float32),
        ] + [pltpu.SemaphoreType.DMA] * (2 * NBUF),
        compiler_params=pltpu.CompilerParams(use_tc_tiling_on_sc=False),
    )
    return k(table, tokens2d, pos)


def kernel(tokens, token_embedding, positional_embedding):
    tokens2d = tokens.astype(jnp.int32).reshape(ROWS // CHUNK, CHUNK)
    out = _emb(tokens2d, token_embedding, positional_embedding)
    return out.reshape(BATCH, SEQ, EMB)


# speedup vs baseline: 2.7686x; 1.2177x over previous
import jax
import jax.numpy as jnp
from jax import lax
from jax.experimental import pallas as pl
from jax.experimental.pallas import tpu as pltpu
from jax.experimental.pallas import tpu_sc as plsc

VOCAB = 100000
EMB = 128
SEQ = 200
BATCH = 1024

NC = 2
NS = 16
NW = NC * NS

ROWS = BATCH * SEQ
ROWS_PER_W = ROWS // NW
CHUNK = 100
CHUNKS_PER_W = ROWS_PER_W // CHUNK
NBUF = 4
GROUPS = CHUNKS_PER_W // NBUF
LANES = 16
VECS_PER_ROW = EMB // LANES


def _body(table_hbm, tokens_hbm, pos_hbm, out_hbm, idx_v, pos_v, rows4, *sems):
    gsem = sems[0:NBUF]
    osem = sems[NBUF:2 * NBUF]
    wid = lax.axis_index("s") * NC + lax.axis_index("c")
    out_base = wid * ROWS_PER_W

    pltpu.sync_copy(tokens_hbm.at[pl.ds(wid * CHUNKS_PER_W, CHUNKS_PER_W)], idx_v)
    pltpu.sync_copy(pos_hbm, pos_v)

    def gather(c, u):
        pltpu.async_copy(table_hbm.at[idx_v.at[c]], rows4.at[u], gsem[u])

    gather(0, 0)
    gather(1, 1)

    def group_step(g, carry):
        for u in range(NBUF):
            c = g * NBUF + u
            u2 = (u + 2) % NBUF
            rows_u = rows4.at[u]
            pltpu.make_async_copy(
                table_hbm.at[idx_v.at[c]], rows_u, gsem[u]).wait()
            @pl.when(c >= 2)
            def _():
                pltpu.make_async_copy(
                    rows4.at[u2], out_hbm.at[pl.ds(0, CHUNK)], osem[u2]).wait()

            @pl.when(c + 2 < CHUNKS_PER_W)
            def _():
                gather(c + 2, u2)

            pr0 = lax.rem(c, 2) * CHUNK

            def add_row(r, carry2):
                pr = pr0 + r
                for d in range(VECS_PER_ROW):
                    sl = pl.ds(d * LANES, LANES)
                    plsc.addupdate(rows_u.at[r, sl], pos_v[pr, sl])
                return carry2

            lax.fori_loop(0, CHUNK, add_row, 0)
            pltpu.async_copy(
                rows_u, out_hbm.at[pl.ds(out_base + c * CHUNK, CHUNK)], osem[u])
        return carry

    lax.fori_loop(0, GROUPS, group_step, 0)

    for u in (2, 3):
        pltpu.make_async_copy(
            rows4.at[u], out_hbm.at[pl.ds(0, CHUNK)], osem[u]).wait()


@jax.jit
def _emb(tokens2d, table, pos):
    mesh = plsc.VectorSubcoreMesh(core_axis_name="c", subcore_axis_name="s")
    k = pl.kernel(
        _body,
        out_type=jax.ShapeDtypeStruct((ROWS, EMB), jnp.float32),
        mesh=mesh,
        scratch_types=[
            pltpu.VMEM((CHUNKS_PER_W, CHUNK), jnp.int32),
            pltpu.VMEM((SEQ, EMB), jnp.float32),
            pltpu.VMEM((NBUF, CHUNK, EMB), jnp.float32),
        ] + [pltpu.SemaphoreType.DMA] * (2 * NBUF),
        compiler_params=pltpu.CompilerParams(use_tc_tiling_on_sc=False),
    )
    return k(table, tokens2d, pos)


def kernel(tokens, token_embedding, positional_embedding):
    tokens2d = tokens.astype(jnp.int32).reshape(ROWS // CHUNK, CHUNK)
    out = _emb(tokens2d, token_embedding, positional_embedding)
    return out.reshape(BATCH, SEQ, EMB)


# speedup vs baseline: 2.7758x; 1.0026x over previous
import jax
import jax.numpy as jnp
from jax import lax
from jax.experimental import pallas as pl
from jax.experimental.pallas import tpu as pltpu
from jax.experimental.pallas import tpu_sc as plsc

VOCAB = 100000
EMB = 128
SEQ = 200
BATCH = 1024

NC = 2
NS = 16
NW = NC * NS

ROWS = BATCH * SEQ
ROWS_PER_W = ROWS // NW
CHUNK = 100
CHUNKS_PER_W = ROWS_PER_W // CHUNK
NBUF = 6
AHEAD = 3
MAIN_GROUPS = 10
LANES = 16
VECS_PER_ROW = EMB // LANES


def _body(table_hbm, tokens_hbm, pos_hbm, out_hbm, idx_v, pos_v, rows6, *sems):
    gsem = sems[0:NBUF]
    osem = sems[NBUF:2 * NBUF]
    wid = lax.axis_index("s") * NC + lax.axis_index("c")
    out_base = wid * ROWS_PER_W

    pltpu.sync_copy(tokens_hbm.at[pl.ds(wid * CHUNKS_PER_W, CHUNKS_PER_W)], idx_v)
    pltpu.sync_copy(pos_hbm, pos_v)

    def gather(c, u):
        pltpu.async_copy(table_hbm.at[idx_v.at[c]], rows6.at[u], gsem[u])

    def wait_gather(c, u):
        pltpu.make_async_copy(table_hbm.at[idx_v.at[c]], rows6.at[u], gsem[u]).wait()

    def wait_out(u):
        pltpu.make_async_copy(rows6.at[u], out_hbm.at[pl.ds(0, CHUNK)], osem[u]).wait()

    def add_pos_and_writeback(c, u):
        rows_u = rows6.at[u]
        pr0 = lax.rem(c, 2) * CHUNK

        def add_row(r, carry2):
            pr = pr0 + r
            for d in range(VECS_PER_ROW):
                sl = pl.ds(d * LANES, LANES)
                plsc.addupdate(rows_u.at[r, sl], pos_v[pr, sl])
            return carry2

        lax.fori_loop(0, CHUNK, add_row, 0)
        pltpu.async_copy(
            rows_u, out_hbm.at[pl.ds(out_base + c * CHUNK, CHUNK)], osem[u])

    for c0 in range(AHEAD):
        gather(c0, c0)

    def group_step(g, carry):
        for u in range(NBUF):
            c = g * NBUF + u
            u3 = (u + AHEAD) % NBUF
            wait_gather(c, u)
            @pl.when(c >= AHEAD)
            def _():
                wait_out(u3)

            @pl.when(c + AHEAD < CHUNKS_PER_W)
            def _():
                gather(c + AHEAD, u3)

            add_pos_and_writeback(c, u)
        return carry

    lax.fori_loop(0, MAIN_GROUPS, group_step, 0)

    for c in range(MAIN_GROUPS * NBUF, CHUNKS_PER_W):
        u = c % NBUF
        u3 = (u + AHEAD) % NBUF
        wait_gather(c, u)
        wait_out(u3)
        if c + AHEAD < CHUNKS_PER_W:
            gather(c + AHEAD, u3)
        add_pos_and_writeback(c, u)

    for c in range(CHUNKS_PER_W - AHEAD, CHUNKS_PER_W):
        wait_out(c % NBUF)


@jax.jit
def _emb(tokens2d, table, pos):
    mesh = plsc.VectorSubcoreMesh(core_axis_name="c", subcore_axis_name="s")
    k = pl.kernel(
        _body,
        out_type=jax.ShapeDtypeStruct((ROWS, EMB), jnp.float32),
        mesh=mesh,
        scratch_types=[
            pltpu.VMEM((CHUNKS_PER_W, CHUNK), jnp.int32),
            pltpu.VMEM((SEQ, EMB), jnp.float32),
            pltpu.VMEM((NBUF, CHUNK, EMB), jnp.float32),
        ] + [pltpu.SemaphoreType.DMA] * (2 * NBUF),
        compiler_params=pltpu.CompilerParams(use_tc_tiling_on_sc=False),
    )
    return k(table, tokens2d, pos)


def kernel(tokens, token_embedding, positional_embedding):
    tokens2d = tokens.astype(jnp.int32).reshape(ROWS // CHUNK, CHUNK)
    out = _emb(tokens2d, token_embedding, positional_embedding)
    return out.reshape(BATCH, SEQ, EMB)
